# baseline (speedup 1.0000x reference)
import functools

import jax
import jax.numpy as jnp
from jax import lax
from jax.experimental import pallas as pl
from jax.experimental.pallas import tpu as pltpu
from jax.experimental.pallas import tpu_sc as plsc

_RADIUS2 = 0.25
_S = 64
_NW = 32


def _sc_ball_group(xyz_flat, q_pad):
    G, N3 = xyz_flat.shape
    N = N3 // 3
    NV = N // 16
    g_per_w = G // _NW
    mesh = plsc.VectorSubcoreMesh(core_axis_name="c", subcore_axis_name="s")

    @functools.partial(
        pl.kernel,
        mesh=mesh,
        compiler_params=pltpu.CompilerParams(needs_layout_passes=False),
        out_type=jax.ShapeDtypeStruct((G, 192), jnp.float32),
        scratch_types=[
            pltpu.VMEM((N * 3,), jnp.float32),
            pltpu.VMEM((16,), jnp.float32),
            pltpu.VMEM((_S,), jnp.int32),
            pltpu.VMEM((192,), jnp.float32),
            pltpu.SMEM((1,), jnp.int32),
        ],
    )
    def k(xyz_hbm, q_hbm, out_hbm, pbuf, qbuf, idxb, gout, cnt_ref):
        wid = lax.axis_index("s") * 2 + lax.axis_index("c")
        iota = lax.iota(jnp.int32, 16)
        zero16 = jnp.zeros((16,), jnp.int32)
        NCHUNK = 4
        CV = NV // NCHUNK
        CW = N3 // NCHUNK

        def per_group(gi, carry):
            g = wid + _NW * gi
            pltpu.sync_copy(q_hbm.at[g], qbuf)
            qrow = qbuf[pl.ds(0, 16)]
            qx = qrow[0]
            qy = qrow[1]
            qz = qrow[2]
            for kb in range(4):
                idxb[pl.ds(kb * 16, 16)] = zero16
            cnt_ref[0] = 0

            def scan_body(c):
                i, cnt = c
                base = i * 16 + iota
                i3 = base * 3
                xv = plsc.load_gather(pbuf, [i3])
                yv = plsc.load_gather(pbuf, [i3 + 1])
                zv = plsc.load_gather(pbuf, [i3 + 2])
                dx = xv - qx
                dy = yv - qy
                dz = zv - qz
                d2 = dx * dx + dy * dy + dz * dz
                m = d2 < _RADIUS2
                pc = plsc.all_reduce_population_count(m)[0]

                @pl.when(pc > 0)
                def _():
                    pos = cnt + jnp.cumsum(m.astype(jnp.int32)) - 1
                    wm = jnp.logical_and(m, pos < _S)
                    plsc.store_scatter(idxb, [pos], base, mask=wm)

                return i + 1, cnt + pc

            for ch in range(NCHUNK):
                @pl.when(cnt_ref[0] < _S)
                def _(ch=ch):
                    pltpu.sync_copy(
                        xyz_hbm.at[g, pl.ds(ch * CW, CW)],
                        pbuf.at[pl.ds(ch * CW, CW)])

                    def cond(c):
                        i, cnt = c
                        return jnp.logical_and(i < (ch + 1) * CV, cnt < _S)

                    _, cnt_out = lax.while_loop(
                        cond, scan_body, (jnp.int32(ch * CV), cnt_ref[0]))
                    cnt_ref[0] = cnt_out

            cnt = cnt_ref[0]

            first = idxb[pl.ds(0, 16)][0]
            for kb in range(4):
                cur = idxb[pl.ds(kb * 16, 16)]
                pos = kb * 16 + iota
                sel = jnp.where(pos < cnt, cur, first)
                s3 = sel * 3
                gx = plsc.load_gather(pbuf, [s3]) - qx
                gy = plsc.load_gather(pbuf, [s3 + 1]) - qy
                gz = plsc.load_gather(pbuf, [s3 + 2]) - qz
                gout[pl.ds(kb * 16, 16)] = gx
                gout[pl.ds(64 + kb * 16, 16)] = gy
                gout[pl.ds(128 + kb * 16, 16)] = gz
            pltpu.sync_copy(gout, out_hbm.at[g])
            return carry

        lax.fori_loop(0, g_per_w, per_group, jnp.int32(0))

    return k(xyz_flat, q_pad)


def _safe_norm_t(x, axis=-1):
    s = jnp.sum(x * x, axis=axis)
    safe = jnp.where(s > 0, s, 1.0)
    return jnp.where(s > 0, jnp.sqrt(safe), 0.0)


def kernel(pcl_coord, joint_coord):
    B, J, N, _ = pcl_coord.shape
    G = B * J
    xyz_flat = pcl_coord.reshape(G, N * 3)
    q = joint_coord.reshape(G, 3)
    q_pad = jnp.concatenate([q, jnp.zeros((G, 13), jnp.float32)], axis=1)

    flat = _sc_ball_group(xyz_flat, q_pad)
    grouped = jnp.transpose(flat.reshape(G, 3, _S), (0, 2, 1))

    v1 = grouped[:, :, None, :] - grouped[:, None, :, :]
    v2 = -v1
    s1 = jnp.sum(v1 * v1, axis=-1)
    v1_norm = jnp.where(s1 > 0, jnp.sqrt(jnp.where(s1 > 0, s1, 1.0)), 0.0)
    den = v1_norm * v1_norm + 1e-06
    cos_angle = (-s1) / den
    sin_angle = _safe_norm_t(jnp.cross(v1, v2), axis=-1) / den
    cot_alpha = cos_angle / (sin_angle + 1e-06)
    laplacian = jnp.sum(v1 * cot_alpha[..., None] / 2, axis=(1, 2))
    curvature_dir = laplacian
    curvature_magnitude = _safe_norm_t(curvature_dir, axis=-1)[..., None]
    curvature_dir_normalized = curvature_dir / (curvature_magnitude + 1e-06)
    curvature_dir_normalized = curvature_dir_normalized.reshape(B, J, 3)
    curvature_magnitude = curvature_magnitude.reshape(B, J)[..., None]
    curvature_vector = curvature_dir_normalized * curvature_magnitude
    return curvature_vector


# speedup vs baseline: 1.0394x; 1.0394x over previous
import functools

import jax
import jax.numpy as jnp
from jax import lax
from jax.experimental import pallas as pl
from jax.experimental.pallas import tpu as pltpu
from jax.experimental.pallas import tpu_sc as plsc

_RADIUS2 = 0.25
_S = 64
_NW = 32


def _sc_ball_group(xyz_flat, q_pad):
    G, N3 = xyz_flat.shape
    N = N3 // 3
    NV = N // 16
    g_per_w = G // _NW
    mesh = plsc.VectorSubcoreMesh(core_axis_name="c", subcore_axis_name="s")

    @functools.partial(
        pl.kernel,
        mesh=mesh,
        compiler_params=pltpu.CompilerParams(needs_layout_passes=False),
        out_type=jax.ShapeDtypeStruct((G, 192), jnp.float32),
        scratch_types=[
            pltpu.VMEM((N * 3,), jnp.float32),
            pltpu.VMEM((16,), jnp.float32),
            pltpu.VMEM((_S,), jnp.int32),
            pltpu.VMEM((192,), jnp.float32),
            pltpu.SMEM((1,), jnp.int32),
        ],
    )
    def k(xyz_hbm, q_hbm, out_hbm, pbuf, qbuf, idxb, gout, cnt_ref):
        wid = lax.axis_index("s") * 2 + lax.axis_index("c")
        iota = lax.iota(jnp.int32, 16)
        zero16 = jnp.zeros((16,), jnp.int32)
        NCHUNK = 4
        CV = NV // NCHUNK
        CW = N3 // NCHUNK

        def per_group(gi, carry):
            g = wid + _NW * gi
            pltpu.sync_copy(q_hbm.at[g], qbuf)
            qrow = qbuf[pl.ds(0, 16)]
            qx = qrow[0]
            qy = qrow[1]
            qz = qrow[2]
            for kb in range(4):
                idxb[pl.ds(kb * 16, 16)] = zero16
            cnt_ref[0] = 0

            def scan_body(c):
                i, cnt = c
                base = i * 16 + iota
                i3 = base * 3
                xv = plsc.load_gather(pbuf, [i3])
                yv = plsc.load_gather(pbuf, [i3 + 1])
                zv = plsc.load_gather(pbuf, [i3 + 2])
                dx = xv - qx
                dy = yv - qy
                dz = zv - qz
                d2 = dx * dx + dy * dy + dz * dz
                m = d2 < _RADIUS2
                pc = plsc.all_reduce_population_count(m)[0]

                @pl.when(pc > 0)
                def _():
                    pos = cnt + jnp.cumsum(m.astype(jnp.int32)) - 1
                    wm = jnp.logical_and(m, pos < _S)
                    plsc.store_scatter(idxb, [pos], base, mask=wm)

                return i + 1, cnt + pc

            pltpu.sync_copy(xyz_hbm.at[g], pbuf)

            def cond(c):
                i, cnt = c
                return jnp.logical_and(i < NV, cnt < _S)

            _, cnt = lax.while_loop(
                cond, scan_body, (jnp.int32(0), jnp.int32(0)))

            first = idxb[pl.ds(0, 16)][0]
            for kb in range(4):
                cur = idxb[pl.ds(kb * 16, 16)]
                pos = kb * 16 + iota
                sel = jnp.where(pos < cnt, cur, first)
                s3 = sel * 3
                gx = plsc.load_gather(pbuf, [s3]) - qx
                gy = plsc.load_gather(pbuf, [s3 + 1]) - qy
                gz = plsc.load_gather(pbuf, [s3 + 2]) - qz
                gout[pl.ds(kb * 16, 16)] = gx
                gout[pl.ds(64 + kb * 16, 16)] = gy
                gout[pl.ds(128 + kb * 16, 16)] = gz
            pltpu.sync_copy(gout, out_hbm.at[g])
            return carry

        lax.fori_loop(0, g_per_w, per_group, jnp.int32(0))

    return k(xyz_flat, q_pad)


def _safe_norm_t(x, axis=-1):
    s = jnp.sum(x * x, axis=axis)
    safe = jnp.where(s > 0, s, 1.0)
    return jnp.where(s > 0, jnp.sqrt(safe), 0.0)


def kernel(pcl_coord, joint_coord):
    B, J, N, _ = pcl_coord.shape
    G = B * J
    xyz_flat = pcl_coord.reshape(G, N * 3)
    q = joint_coord.reshape(G, 3)
    q_pad = jnp.concatenate([q, jnp.zeros((G, 13), jnp.float32)], axis=1)

    flat = _sc_ball_group(xyz_flat, q_pad)
    grouped = jnp.transpose(flat.reshape(G, 3, _S), (0, 2, 1))

    v1 = grouped[:, :, None, :] - grouped[:, None, :, :]
    v2 = -v1
    s1 = jnp.sum(v1 * v1, axis=-1)
    v1_norm = jnp.where(s1 > 0, jnp.sqrt(jnp.where(s1 > 0, s1, 1.0)), 0.0)
    den = v1_norm * v1_norm + 1e-06
    cos_angle = (-s1) / den
    sin_angle = _safe_norm_t(jnp.cross(v1, v2), axis=-1) / den
    cot_alpha = cos_angle / (sin_angle + 1e-06)
    laplacian = jnp.sum(v1 * cot_alpha[..., None] / 2, axis=(1, 2))
    curvature_dir = laplacian
    curvature_magnitude = _safe_norm_t(curvature_dir, axis=-1)[..., None]
    curvature_dir_normalized = curvature_dir / (curvature_magnitude + 1e-06)
    curvature_dir_normalized = curvature_dir_normalized.reshape(B, J, 3)
    curvature_magnitude = curvature_magnitude.reshape(B, J)[..., None]
    curvature_vector = curvature_dir_normalized * curvature_magnitude
    return curvature_vector


# speedup vs baseline: 1.1100x; 1.0679x over previous
import functools

import jax
import jax.numpy as jnp
from jax import lax
from jax.experimental import pallas as pl
from jax.experimental.pallas import tpu as pltpu
from jax.experimental.pallas import tpu_sc as plsc

_RADIUS2 = 0.25
_S = 64
_NW = 32


def _sc_ball_group(xyz_flat, q_sc):
    G, N3 = xyz_flat.shape
    N = N3 // 3
    NV = N // 16
    g_per_w = G // _NW
    NP = g_per_w // 2
    mesh = plsc.VectorSubcoreMesh(core_axis_name="c", subcore_axis_name="s")

    @functools.partial(
        pl.kernel,
        mesh=mesh,
        compiler_params=pltpu.CompilerParams(needs_layout_passes=False),
        out_type=jax.ShapeDtypeStruct((_NW, 192 * g_per_w), jnp.float32),
        scratch_types=[
            pltpu.VMEM((N3,), jnp.float32),
            pltpu.VMEM((N3,), jnp.float32),
            pltpu.VMEM((16 * g_per_w,), jnp.float32),
            pltpu.VMEM((_S,), jnp.int32),
            pltpu.VMEM((192 * g_per_w,), jnp.float32),
            pltpu.SemaphoreType.DMA,
            pltpu.SemaphoreType.DMA,
        ],
    )
    def k(xyz_hbm, q_hbm, out_hbm, buf_a, buf_b, qall, idxb, outall,
          sem_a, sem_b):
        wid = lax.axis_index("s") * 2 + lax.axis_index("c")
        iota = lax.iota(jnp.int32, 16)
        zero16 = jnp.zeros((16,), jnp.int32)

        pltpu.sync_copy(q_hbm.at[wid], qall)
        pltpu.async_copy(xyz_hbm.at[wid], buf_a, sem_a)

        def process(gi, pbuf):
            qrow = qall[pl.ds(gi * 16, 16)]
            qx = qrow[0]
            qy = qrow[1]
            qz = qrow[2]
            for kb in range(4):
                idxb[pl.ds(kb * 16, 16)] = zero16

            def cond(c):
                i, cnt = c
                return jnp.logical_and(i < NV, cnt < _S)

            def body(c):
                i, cnt = c
                base = i * 16 + iota
                i3 = base * 3
                xv = plsc.load_gather(pbuf, [i3])
                yv = plsc.load_gather(pbuf, [i3 + 1])
                zv = plsc.load_gather(pbuf, [i3 + 2])
                dx = xv - qx
                dy = yv - qy
                dz = zv - qz
                d2 = dx * dx + dy * dy + dz * dz
                m = d2 < _RADIUS2
                mi = m.astype(jnp.int32)
                pos = cnt + jnp.cumsum(mi) - 1
                wm = jnp.logical_and(m, pos < _S)
                plsc.store_scatter(idxb, [pos], base, mask=wm)
                return i + 1, cnt + jnp.sum(mi)

            _, cnt = lax.while_loop(cond, body, (jnp.int32(0), jnp.int32(0)))

            first = idxb[pl.ds(0, 16)][0]
            obase = gi * 192
            for kb in range(4):
                cur = idxb[pl.ds(kb * 16, 16)]
                pos = kb * 16 + iota
                sel = jnp.where(pos < cnt, cur, first)
                s3 = sel * 3
                gx = plsc.load_gather(pbuf, [s3]) - qx
                gy = plsc.load_gather(pbuf, [s3 + 1]) - qy
                gz = plsc.load_gather(pbuf, [s3 + 2]) - qz
                outall[pl.ds(obase + kb * 16, 16)] = gx
                outall[pl.ds(obase + 64 + kb * 16, 16)] = gy
                outall[pl.ds(obase + 128 + kb * 16, 16)] = gz

        def per_pair(pi, carry):
            g0 = wid + 64 * pi
            g1 = g0 + 32
            cp_b = pltpu.async_copy(xyz_hbm.at[g1], buf_b, sem_b)
            pltpu.make_async_copy(xyz_hbm.at[g0], buf_a, sem_a).wait()
            process(2 * pi, buf_a)

            @pl.when(pi < NP - 1)
            def _():
                pltpu.async_copy(xyz_hbm.at[g0 + 64], buf_a, sem_a)

            cp_b.wait()
            process(2 * pi + 1, buf_b)
            return carry

        lax.fori_loop(0, NP, per_pair, jnp.int32(0))
        pltpu.sync_copy(outall, out_hbm.at[wid])

    return k(xyz_flat, q_sc)


def _safe_norm_t(x, axis=-1):
    s = jnp.sum(x * x, axis=axis)
    safe = jnp.where(s > 0, s, 1.0)
    return jnp.where(s > 0, jnp.sqrt(safe), 0.0)


def kernel(pcl_coord, joint_coord):
    B, J, N, _ = pcl_coord.shape
    G = B * J
    g_per_w = G // _NW
    xyz_flat = pcl_coord.reshape(G, N * 3)
    q = joint_coord.reshape(G, 3)
    q_pad = jnp.concatenate([q, jnp.zeros((G, 13), jnp.float32)], axis=1)
    q_sc = jnp.transpose(q_pad.reshape(g_per_w, _NW, 16), (1, 0, 2))
    q_sc = q_sc.reshape(_NW, 16 * g_per_w)

    flat = _sc_ball_group(xyz_flat, q_sc)
    flat = flat.reshape(_NW, g_per_w, 192)
    flat = jnp.transpose(flat, (1, 0, 2)).reshape(G, 192)
    grouped = jnp.transpose(flat.reshape(G, 3, _S), (0, 2, 1))

    v1 = grouped[:, :, None, :] - grouped[:, None, :, :]
    v2 = -v1
    s1 = jnp.sum(v1 * v1, axis=-1)
    v1_norm = jnp.where(s1 > 0, jnp.sqrt(jnp.where(s1 > 0, s1, 1.0)), 0.0)
    den = v1_norm * v1_norm + 1e-06
    cos_angle = (-s1) / den
    sin_angle = _safe_norm_t(jnp.cross(v1, v2), axis=-1) / den
    cot_alpha = cos_angle / (sin_angle + 1e-06)
    laplacian = jnp.sum(v1 * cot_alpha[..., None] / 2, axis=(1, 2))
    curvature_dir = laplacian
    curvature_magnitude = _safe_norm_t(curvature_dir, axis=-1)[..., None]
    curvature_dir_normalized = curvature_dir / (curvature_magnitude + 1e-06)
    curvature_dir_normalized = curvature_dir_normalized.reshape(B, J, 3)
    curvature_magnitude = curvature_magnitude.reshape(B, J)[..., None]
    curvature_vector = curvature_dir_normalized * curvature_magnitude
    return curvature_vector


# speedup vs baseline: 1.1988x; 1.0799x over previous
import functools

import jax
import jax.numpy as jnp
from jax import lax
from jax.experimental import pallas as pl
from jax.experimental.pallas import tpu as pltpu
from jax.experimental.pallas import tpu_sc as plsc

_RADIUS2 = 0.25
_S = 64
_NW = 32


def _sc_ball_group(xyz_flat, q_sc):
    G, N3 = xyz_flat.shape
    N = N3 // 3
    NV = N // 16
    g_per_w = G // _NW
    NP = g_per_w // 2
    mesh = plsc.VectorSubcoreMesh(core_axis_name="c", subcore_axis_name="s")

    @functools.partial(
        pl.kernel,
        mesh=mesh,
        compiler_params=pltpu.CompilerParams(needs_layout_passes=False),
        out_type=jax.ShapeDtypeStruct((_NW, 192 * g_per_w), jnp.float32),
        scratch_types=[
            pltpu.VMEM((N3,), jnp.float32),
            pltpu.VMEM((N3,), jnp.float32),
            pltpu.VMEM((16 * g_per_w,), jnp.float32),
            pltpu.VMEM((_S,), jnp.int32),
            pltpu.VMEM((192 * g_per_w,), jnp.float32),
            pltpu.SemaphoreType.DMA,
            pltpu.SemaphoreType.DMA,
        ],
    )
    def k(xyz_hbm, q_hbm, out_hbm, buf_a, buf_b, qall, idxb, outall,
          sem_a, sem_b):
        wid = lax.axis_index("s") * 2 + lax.axis_index("c")
        iota = lax.iota(jnp.int32, 16)
        zero16 = jnp.zeros((16,), jnp.int32)

        pltpu.sync_copy(q_hbm.at[wid], qall)
        pltpu.async_copy(xyz_hbm.at[wid], buf_a, sem_a)

        def process(gi, pbuf):
            qrow = qall[pl.ds(gi * 16, 16)]
            qx = qrow[0]
            qy = qrow[1]
            qz = qrow[2]
            for kb in range(4):
                idxb[pl.ds(kb * 16, 16)] = zero16

            def cond(c):
                i, cnt = c
                return jnp.logical_and(i < NV, cnt < _S)

            def body(c):
                i, cnt = c
                for u in range(2):
                    base = (i + u) * 16 + iota
                    i3 = base * 3
                    xv = plsc.load_gather(pbuf, [i3])
                    yv = plsc.load_gather(pbuf, [i3 + 1])
                    zv = plsc.load_gather(pbuf, [i3 + 2])
                    dx = xv - qx
                    dy = yv - qy
                    dz = zv - qz
                    d2 = dx * dx + dy * dy + dz * dz
                    m = d2 < _RADIUS2
                    mi = m.astype(jnp.int32)
                    pos = cnt + jnp.cumsum(mi) - 1
                    wm = jnp.logical_and(m, pos < _S)
                    plsc.store_scatter(idxb, [pos], base, mask=wm)
                    cnt = cnt + jnp.sum(mi)
                return i + 2, cnt

            _, cnt = lax.while_loop(cond, body, (jnp.int32(0), jnp.int32(0)))

            first = idxb[pl.ds(0, 16)][0]
            obase = gi * 192
            for kb in range(4):
                cur = idxb[pl.ds(kb * 16, 16)]
                pos = kb * 16 + iota
                sel = jnp.where(pos < cnt, cur, first)
                s3 = sel * 3
                gx = plsc.load_gather(pbuf, [s3]) - qx
                gy = plsc.load_gather(pbuf, [s3 + 1]) - qy
                gz = plsc.load_gather(pbuf, [s3 + 2]) - qz
                outall[pl.ds(obase + kb * 16, 16)] = gx
                outall[pl.ds(obase + 64 + kb * 16, 16)] = gy
                outall[pl.ds(obase + 128 + kb * 16, 16)] = gz

        def per_pair(pi, carry):
            g0 = wid + 64 * pi
            g1 = g0 + 32
            cp_b = pltpu.async_copy(xyz_hbm.at[g1], buf_b, sem_b)
            pltpu.make_async_copy(xyz_hbm.at[g0], buf_a, sem_a).wait()
            process(2 * pi, buf_a)

            @pl.when(pi < NP - 1)
            def _():
                pltpu.async_copy(xyz_hbm.at[g0 + 64], buf_a, sem_a)

            cp_b.wait()
            process(2 * pi + 1, buf_b)
            return carry

        lax.fori_loop(0, NP, per_pair, jnp.int32(0))
        pltpu.sync_copy(outall, out_hbm.at[wid])

    return k(xyz_flat, q_sc)


def _safe_norm_t(x, axis=-1):
    s = jnp.sum(x * x, axis=axis)
    safe = jnp.where(s > 0, s, 1.0)
    return jnp.where(s > 0, jnp.sqrt(safe), 0.0)


def kernel(pcl_coord, joint_coord):
    B, J, N, _ = pcl_coord.shape
    G = B * J
    g_per_w = G // _NW
    xyz_flat = pcl_coord.reshape(G, N * 3)
    q = joint_coord.reshape(G, 3)
    q_pad = jnp.concatenate([q, jnp.zeros((G, 13), jnp.float32)], axis=1)
    q_sc = jnp.transpose(q_pad.reshape(g_per_w, _NW, 16), (1, 0, 2))
    q_sc = q_sc.reshape(_NW, 16 * g_per_w)

    flat = _sc_ball_group(xyz_flat, q_sc)
    flat = flat.reshape(_NW, g_per_w, 192)
    flat = jnp.transpose(flat, (1, 0, 2)).reshape(G, 192)
    grouped = jnp.transpose(flat.reshape(G, 3, _S), (0, 2, 1))

    v1 = grouped[:, :, None, :] - grouped[:, None, :, :]
    v2 = -v1
    s1 = jnp.sum(v1 * v1, axis=-1)
    v1_norm = jnp.where(s1 > 0, jnp.sqrt(jnp.where(s1 > 0, s1, 1.0)), 0.0)
    den = v1_norm * v1_norm + 1e-06
    cos_angle = (-s1) / den
    sin_angle = _safe_norm_t(jnp.cross(v1, v2), axis=-1) / den
    cot_alpha = cos_angle / (sin_angle + 1e-06)
    laplacian = jnp.sum(v1 * cot_alpha[..., None] / 2, axis=(1, 2))
    curvature_dir = laplacian
    curvature_magnitude = _safe_norm_t(curvature_dir, axis=-1)[..., None]
    curvature_dir_normalized = curvature_dir / (curvature_magnitude + 1e-06)
    curvature_dir_normalized = curvature_dir_normalized.reshape(B, J, 3)
    curvature_magnitude = curvature_magnitude.reshape(B, J)[..., None]
    curvature_vector = curvature_dir_normalized * curvature_magnitude
    return curvature_vector
